# baseline (speedup 1.0000x reference)
import functools

import jax
import jax.numpy as jnp
from jax import lax
from jax.experimental import pallas as pl
from jax.experimental.pallas import tpu as pltpu

E = 64
K = 8
_T = 256

_NEG = float("-inf")


def _router_body(x_ref, wt_ref, b_ref, w_out, mask_out, idx_out):
    logits = jnp.dot(x_ref[...], wt_ref[...],
                     preferred_element_type=jnp.float32)
    logits = logits + b_ref[...]
    t = logits.shape[0]
    iota = lax.broadcasted_iota(jnp.int32, (t, E), 1)

    work = logits
    vals = []
    idxs = []
    for _ in range(K):
        m = jnp.max(work, axis=1, keepdims=True)
        sel = jnp.min(jnp.where(work == m, iota, E), axis=1, keepdims=True)
        vals.append(m)
        idxs.append(sel)
        work = jnp.where(iota == sel, _NEG, work)

    v = jnp.concatenate(vals, axis=1)
    ii = jnp.concatenate(idxs, axis=1)
    e = jnp.exp(v - v[:, 0:1])
    w = e / jnp.sum(e, axis=1, keepdims=True)

    w_out[...] = w
    idx_out[...] = ii
    mask = jnp.zeros((t, E), jnp.float32)
    for k in range(K):
        mask = mask + jnp.where(iota == idxs[k], w[:, k:k + 1], 0.0)
    mask_out[...] = mask


@jax.jit
def kernel(x, W, b):
    B, S, D = x.shape
    n = B * S
    xf = x.reshape(n, D)
    wt = W.T
    b2 = b.reshape(1, E)
    grid = n // _T
    w_flat, mask_flat, idx_flat = pl.pallas_call(
        _router_body,
        grid=(grid,),
        in_specs=[
            pl.BlockSpec((_T, D), lambda i: (i, 0)),
            pl.BlockSpec((D, E), lambda i: (0, 0)),
            pl.BlockSpec((1, E), lambda i: (0, 0)),
        ],
        out_specs=[
            pl.BlockSpec((_T, K), lambda i: (i, 0)),
            pl.BlockSpec((_T, E), lambda i: (i, 0)),
            pl.BlockSpec((_T, K), lambda i: (i, 0)),
        ],
        out_shape=[
            jax.ShapeDtypeStruct((n, K), jnp.float32),
            jax.ShapeDtypeStruct((n, E), jnp.float32),
            jax.ShapeDtypeStruct((n, K), jnp.int32),
        ],
        compiler_params=pltpu.CompilerParams(
            dimension_semantics=("parallel",),
        ),
    )(xf, wt, b2)
    return (w_flat.reshape(B, S, K),
            mask_flat.reshape(B, S, E),
            idx_flat.reshape(B, S, K))


# speedup vs baseline: 2.0293x; 2.0293x over previous
import functools

import jax
import jax.numpy as jnp
from jax import lax
from jax.experimental import pallas as pl
from jax.experimental.pallas import tpu as pltpu

E = 64
K = 8
_T = 256

_NEG = float("-inf")


def _router_body(x_ref, w_ref, b_ref, w_out, mask_out, idx_out):
    logits = lax.dot_general(
        w_ref[...], x_ref[...], (((1,), (1,)), ((), ())),
        preferred_element_type=jnp.float32)
    logits = logits + b_ref[...]
    t = logits.shape[1]
    iota_e = lax.broadcasted_iota(jnp.int32, (E, t), 0)

    work = logits
    vals = []
    idxs = []
    for _ in range(K):
        m = jnp.max(work, axis=0, keepdims=True)
        sel = jnp.min(jnp.where(work == m, iota_e, E), axis=0, keepdims=True)
        vals.append(m)
        idxs.append(sel)
        work = jnp.where(iota_e == sel, _NEG, work)

    v = jnp.concatenate(vals, axis=0)
    ii = jnp.concatenate(idxs, axis=0)
    e = jnp.exp(v - v[0:1])
    w = e / jnp.sum(e, axis=0, keepdims=True)

    w_out[...] = w
    idx_out[...] = ii
    mask = jnp.zeros((E, t), jnp.float32)
    for k in range(K):
        mask = mask + jnp.where(iota_e == ii[k:k + 1], w[k:k + 1], 0.0)
    mask_out[...] = mask


@jax.jit
def kernel(x, W, b):
    B, S, D = x.shape
    n = B * S
    xf = x.reshape(n, D)
    b2 = b.reshape(E, 1)
    grid = n // _T
    w_t, mask_t, idx_t = pl.pallas_call(
        _router_body,
        grid=(grid,),
        in_specs=[
            pl.BlockSpec((_T, D), lambda i: (i, 0)),
            pl.BlockSpec((E, D), lambda i: (0, 0)),
            pl.BlockSpec((E, 1), lambda i: (0, 0)),
        ],
        out_specs=[
            pl.BlockSpec((K, _T), lambda i: (0, i)),
            pl.BlockSpec((E, _T), lambda i: (0, i)),
            pl.BlockSpec((K, _T), lambda i: (0, i)),
        ],
        out_shape=[
            jax.ShapeDtypeStruct((K, n), jnp.float32),
            jax.ShapeDtypeStruct((E, n), jnp.float32),
            jax.ShapeDtypeStruct((K, n), jnp.int32),
        ],
        compiler_params=pltpu.CompilerParams(
            dimension_semantics=("parallel",),
        ),
    )(xf, W, b2)
    return (w_t.T.reshape(B, S, K),
            mask_t.T.reshape(B, S, E),
            idx_t.T.reshape(B, S, K))
